# baseline (speedup 1.0000x reference)
import functools
import math

import jax
import jax.numpy as jnp
from jax import lax
from jax.experimental import pallas as pl
from jax.experimental.pallas import tpu as pltpu
from jax.experimental.pallas import tpu_sc as plsc

V = 32000
SMOOTH_VAL = 0.1 / (V - 2)
CONFIDENCE = 0.9
C1 = SMOOTH_VAL * (V - 2) * math.log(SMOOTH_VAL) + CONFIDENCE * math.log(CONFIDENCE)

BR = 128
BC = 32000

NW = 32
RSC = 0


def _tc_body(tgt_ref, out_ref, acc_ref):
    i = pl.program_id(0)
    j = pl.program_id(1)

    @pl.when((i == 0) & (j == 0))
    def _init():
        acc_ref[0, 0] = 0.0

    blk = out_ref[...]
    tcol = tgt_ref[...]
    m = (tcol != 0).astype(jnp.float32)
    rs = jnp.sum(blk, axis=1, keepdims=True)
    part = -SMOOTH_VAL * jnp.sum(rs * m)
    extra = jnp.sum(m * (C1 + SMOOTH_VAL * blk[:, 0:1]))
    part = part + jnp.where(j == 0, extra, 0.0)
    acc_ref[0, 0] += part


def _tc_partial(out2d, tgt2d):
    n = out2d.shape[0]
    rb0 = RSC // BR
    return pl.pallas_call(
        _tc_body,
        grid=((n - RSC) // BR, V // BC),
        in_specs=[
            pl.BlockSpec((BR, 1), lambda i, j: (i + rb0, 0)),
            pl.BlockSpec((BR, BC), lambda i, j: (i + rb0, j)),
        ],
        out_specs=pl.BlockSpec(
            (1, 1), lambda i, j: (0, 0), memory_space=pltpu.SMEM),
        out_shape=jax.ShapeDtypeStruct((1, 1), jnp.float32),
    )(tgt2d, out2d)


def _sc_partial(out2d, tgt):
    n = tgt.shape[0]
    gch = n // NW
    ctiles = V // 128
    mesh = plsc.VectorSubcoreMesh(core_axis_name="c", subcore_axis_name="s")

    @functools.partial(
        pl.kernel,
        mesh=mesh,
        out_type=jax.ShapeDtypeStruct((NW, 16), jnp.float32),
        scratch_types=[
            pltpu.VMEM((gch,), jnp.int32),
            pltpu.VMEM((16, 8, 128), jnp.float32),
            pltpu.VMEM((16,), jnp.float32),
            pltpu.SemaphoreType.DMA,
        ],
    )
    def k(out_hbm, tgt_hbm, o_hbm, gt_v, tiles_v, res_v, gsem):
        wid = lax.axis_index("s") * 2 + lax.axis_index("c")
        gbase = wid * gch
        pltpu.sync_copy(tgt_hbm.at[pl.ds(gbase, gch)], gt_v)
        iot = lax.iota(jnp.int32, 16)
        res_v[...] = jnp.zeros((16,), jnp.float32)
        for q in range(gch // 16):
            t16 = gt_v[pl.ds(q * 16, 16)]
            c16 = t16 >> 7
            handles = []
            for u in range(16):
                g = q * 16 + u
                handles.append(pltpu.async_copy(
                    out_hbm.at[pl.ds(gbase + (g & ~7), 8),
                               pl.ds(c16[u] * 128, 128)],
                    tiles_v.at[u], gsem))
            for h in handles:
                h.wait()
            for u in range(16):
                tu = t16[u]

                @pl.when(tu != 0)
                def _add(u=u, tu=tu):
                    v16 = tiles_v[u, u & 7, pl.ds(((tu >> 4) & 7) * 16, 16)]
                    res_v[...] = res_v[...] + jnp.where(
                        iot == (tu & 15), v16, 0.0)
        res_v[...] = (SMOOTH_VAL - CONFIDENCE) * res_v[...]
        pltpu.sync_copy(res_v, o_hbm.at[wid])

    return k(out2d, tgt)


def kernel(output, target, one_hot):
    n = output.shape[0] * output.shape[1]
    out2d = output.reshape(n, V)
    tgt = target.reshape(n).astype(jnp.int32)
    acc = _tc_partial(out2d, tgt.reshape(n, 1))
    g = _sc_partial(out2d, tgt)
    return acc[0, 0] + jnp.sum(g)
